# baseline (speedup 1.0000x reference)
import functools

import jax
import jax.numpy as jnp
from jax import lax
from jax.experimental import pallas as pl
from jax.experimental.pallas import tpu as pltpu
from jax.experimental.pallas import tpu_sc as plsc

NC = 2
NS = 16
L = 16
NSHIFT = 8


def _sc_gather_rev8(S, D, n_rows):
    max_len = (n_rows + 1) // 2
    offset = max_len - 1
    top = offset + S - 1
    NW = NC * NS
    W2 = 2 * S
    RW = NSHIFT * W2 // NW
    assert RW % L == 0 and W2 % RW == 0

    mesh = plsc.VectorSubcoreMesh(
        core_axis_name="c", subcore_axis_name="s",
        num_cores=NC, num_subcores=NS,
    )

    @functools.partial(
        pl.kernel,
        out_type=jax.ShapeDtypeStruct((NSHIFT * W2, D), jnp.float32),
        mesh=mesh,
        compiler_params=pltpu.CompilerParams(use_tc_tiling_on_sc=False),
        scratch_types=[
            pltpu.VMEM((RW,), jnp.int32),
            pltpu.VMEM((RW, D), jnp.float32),
            pltpu.SemaphoreType.DMA,
        ],
    )
    def gather(pe_hbm, rev8_hbm, idx_v, win_v, sem):
        wid = lax.axis_index("s") * NC + lax.axis_index("c")
        shift = wid // (W2 // RW)
        base = top + (W2 - 1) * shift - wid * RW
        for t in range(RW // L):
            idx_v[pl.ds(t * L, L)] = (
                jnp.full((L,), base - t * L, jnp.int32)
                - lax.iota(jnp.int32, L)
            )
        copies = []
        k = 0
        while k < RW:
            n = min(128, RW - k)
            copies.append(pltpu.async_copy(
                pe_hbm.at[idx_v.at[pl.ds(k, n)]],
                win_v.at[pl.ds(k, n), :],
                sem,
            ))
            k += n
        for cp in copies:
            cp.wait()
        pltpu.sync_copy(win_v, rev8_hbm.at[pl.ds(wid * RW, RW), :])

    return gather


def _tc_broadcast(S, D, rows_per_step=16):
    W2 = 2 * S

    def body(rev_ref, out_ref):
        b = pl.program_id(0)
        for r in range(rows_per_step):
            i = b * rows_per_step + r
            t = (S - 1) - i
            s = lax.rem(t, NSHIFT)
            row = (W2 - 1) * s + t
            row = pl.multiple_of(row, NSHIFT)
            out_ref[r] = rev_ref[pl.ds(row, S), :]

    return pl.pallas_call(
        body,
        grid=(S // rows_per_step,),
        in_specs=[pl.BlockSpec((NSHIFT * W2, D), lambda i: (0, 0))],
        out_specs=pl.BlockSpec((rows_per_step, S, D), lambda i: (i, 0, 0)),
        out_shape=jax.ShapeDtypeStruct((S, S, D), jnp.float32),
        compiler_params=pltpu.CompilerParams(
            dimension_semantics=("arbitrary",),
            vmem_limit_bytes=64 * 1024 * 1024,
        ),
    )


def kernel(x, relative_pe, seq_len):
    S = x.shape[1]
    n_rows, D = relative_pe.shape
    rev8 = _sc_gather_rev8(S, D, n_rows)(relative_pe)
    return _tc_broadcast(S, D)(rev8)


# speedup vs baseline: 1.0947x; 1.0947x over previous
import functools

import jax
import jax.numpy as jnp
from jax import lax
from jax.experimental import pallas as pl
from jax.experimental.pallas import tpu as pltpu
from jax.experimental.pallas import tpu_sc as plsc

NC = 2
NS = 16
L = 16
NSHIFT = 8
LANE = 128


def _sc_gather_rev8(S, D, n_rows):
    max_len = (n_rows + 1) // 2
    offset = max_len - 1
    top = offset + S - 1
    NW = NC * NS
    W2 = 2 * S
    RW = NSHIFT * W2 // NW
    HW = 2 * RW
    n_half = D // LANE
    assert n_half == 2 and RW % NSHIFT == 0 and W2 % RW == 0

    mesh = plsc.VectorSubcoreMesh(
        core_axis_name="c", subcore_axis_name="s",
        num_cores=NC, num_subcores=NS,
    )

    @functools.partial(
        pl.kernel,
        out_type=jax.ShapeDtypeStruct((NSHIFT * W2 * n_half, LANE), jnp.float32),
        mesh=mesh,
        compiler_params=pltpu.CompilerParams(use_tc_tiling_on_sc=False),
        scratch_types=[
            pltpu.VMEM((HW,), jnp.int32),
            pltpu.VMEM((HW, LANE), jnp.float32),
            pltpu.SemaphoreType.DMA,
        ],
    )
    def gather(pe2_hbm, rev_hbm, idx_v, win_v, sem):
        wid = lax.axis_index("s") * NC + lax.axis_index("c")
        shift = wid // (W2 // RW)
        base = 2 * (top + (W2 - 1) * shift - wid * RW)
        lv = lax.iota(jnp.int32, L)
        lane_adj = lax.shift_right_logical(lv, 3) - 2 * lax.bitwise_and(lv, 7)
        for g in range(HW // L):
            idx_v[pl.ds(g * L, L)] = (
                jnp.full((L,), base - 2 * NSHIFT * g, jnp.int32) + lane_adj
            )
        copies = []
        k = 0
        while k < HW:
            n = min(128, HW - k)
            copies.append(pltpu.async_copy(
                pe2_hbm.at[idx_v.at[pl.ds(k, n)]],
                win_v.at[pl.ds(k, n), :],
                sem,
            ))
            k += n
        for cp in copies:
            cp.wait()
        pltpu.sync_copy(win_v, rev_hbm.at[pl.ds(wid * HW, HW), :])

    return gather


def _tc_broadcast(S, D, rows_per_step=8):
    W2 = 2 * S
    n_half = D // LANE

    def body(rev_ref, out_ref):
        b = pl.program_id(0)
        for r in range(rows_per_step):
            i = b * rows_per_step + r
            t = (S - 1) - i
            s = lax.rem(t, NSHIFT)
            row = (W2 - 1) * s + t
            row = pl.multiple_of(row, NSHIFT)
            rt = row // NSHIFT
            for ct in range(n_half):
                blk = rev_ref[pl.ds(rt, S // NSHIFT), ct, :, :]
                out_ref[r, :, ct * LANE:(ct + 1) * LANE] = (
                    blk.reshape(S, LANE)
                )

    return pl.pallas_call(
        body,
        grid=(S // rows_per_step,),
        in_specs=[pl.BlockSpec(
            (NSHIFT * W2 // NSHIFT, n_half, NSHIFT, LANE),
            lambda i: (0, 0, 0, 0),
        )],
        out_specs=pl.BlockSpec((rows_per_step, S, D), lambda i: (i, 0, 0)),
        out_shape=jax.ShapeDtypeStruct((S, S, D), jnp.float32),
        compiler_params=pltpu.CompilerParams(
            dimension_semantics=("arbitrary",),
            vmem_limit_bytes=64 * 1024 * 1024,
        ),
    )


def kernel(x, relative_pe, seq_len):
    S = x.shape[1]
    n_rows, D = relative_pe.shape
    pe2 = relative_pe.reshape(n_rows * (D // LANE), LANE)
    rev2d = _sc_gather_rev8(S, D, n_rows)(pe2)
    rev4 = rev2d.reshape(NSHIFT * 2 * S // NSHIFT, D // LANE, NSHIFT, LANE)
    return _tc_broadcast(S, D)(rev4)


# speedup vs baseline: 1.1696x; 1.0685x over previous
import functools

import jax
import jax.numpy as jnp
from jax import lax
from jax.experimental import pallas as pl
from jax.experimental.pallas import tpu as pltpu
from jax.experimental.pallas import tpu_sc as plsc

NC = 2
NS = 16
L = 16
NSHIFT = 8
LANE = 128


def _used_window(S, n_rows):
    max_len = (n_rows + 1) // 2
    offset = max_len - 1
    top = offset + S - 1
    lo = max(0, top - (2 * S - 1) - (NSHIFT - 1))
    lo -= lo % NSHIFT
    return lo, top + 1, top


def _sc_gather_rev8(S, D, n_rows, lo, top):
    NW = NC * NS
    W2 = 2 * S
    RW = NSHIFT * W2 // NW
    HW = 2 * RW
    n_half = D // LANE
    assert n_half == 2 and RW % NSHIFT == 0 and W2 % RW == 0

    mesh = plsc.VectorSubcoreMesh(
        core_axis_name="c", subcore_axis_name="s",
        num_cores=NC, num_subcores=NS,
    )

    @functools.partial(
        pl.kernel,
        out_type=jax.ShapeDtypeStruct((NSHIFT * W2 * n_half, LANE), jnp.float32),
        mesh=mesh,
        compiler_params=pltpu.CompilerParams(use_tc_tiling_on_sc=False),
        scratch_types=[
            pltpu.VMEM((HW,), jnp.int32),
            pltpu.VMEM((HW, LANE), jnp.float32),
            pltpu.SemaphoreType.DMA,
        ],
    )
    def gather(pe2_hbm, rev_hbm, idx_v, win_v, sem):
        wid = lax.axis_index("s") * NC + lax.axis_index("c")
        shift = wid // (W2 // RW)
        base = 2 * (top - lo + (W2 - 1) * shift - wid * RW)
        lv = lax.iota(jnp.int32, L)
        lane_adj = lax.shift_right_logical(lv, 3) - 2 * lax.bitwise_and(lv, 7)
        for g in range(HW // L):
            idx_v[pl.ds(g * L, L)] = (
                jnp.full((L,), base - 2 * NSHIFT * g, jnp.int32) + lane_adj
            )
        copies = []
        k = 0
        while k < HW:
            n = min(128, HW - k)
            copies.append(pltpu.async_copy(
                pe2_hbm.at[idx_v.at[pl.ds(k, n)]],
                win_v.at[pl.ds(k, n), :],
                sem,
            ))
            k += n
        for cp in copies:
            cp.wait()
        pltpu.sync_copy(win_v, rev_hbm.at[pl.ds(wid * HW, HW), :])

    return gather


def _tc_broadcast(S, D, rows_per_step=8):
    W2 = 2 * S
    n_half = D // LANE

    def body(rev_ref, out_ref):
        b = pl.program_id(0)
        for r in range(rows_per_step):
            i = b * rows_per_step + r
            t = (S - 1) - i
            s = lax.rem(t, NSHIFT)
            row = (W2 - 1) * s + t
            row = pl.multiple_of(row, NSHIFT)
            rt = row // NSHIFT
            for ct in range(n_half):
                blk = rev_ref[pl.ds(rt, S // NSHIFT), ct, :, :]
                out_ref[r, :, ct * LANE:(ct + 1) * LANE] = (
                    blk.reshape(S, LANE)
                )

    return pl.pallas_call(
        body,
        grid=(S // rows_per_step,),
        in_specs=[pl.BlockSpec(
            (NSHIFT * W2 // NSHIFT, n_half, NSHIFT, LANE),
            lambda i: (0, 0, 0, 0),
        )],
        out_specs=pl.BlockSpec((rows_per_step, S, D), lambda i: (i, 0, 0)),
        out_shape=jax.ShapeDtypeStruct((S, S, D), jnp.float32),
        compiler_params=pltpu.CompilerParams(
            dimension_semantics=("arbitrary",),
            vmem_limit_bytes=64 * 1024 * 1024,
        ),
    )


def kernel(x, relative_pe, seq_len):
    S = x.shape[1]
    n_rows, D = relative_pe.shape
    lo, hi, top = _used_window(S, n_rows)
    pe_win = lax.slice(relative_pe, (lo, 0), (hi, D))
    pe2 = pe_win.reshape((hi - lo) * (D // LANE), LANE)
    rev2d = _sc_gather_rev8(S, D, n_rows, lo, top)(pe2)
    rev4 = rev2d.reshape(NSHIFT * 2 * S // NSHIFT, D // LANE, NSHIFT, LANE)
    return _tc_broadcast(S, D)(rev4)
